# baseline (speedup 1.0000x reference)
import jax
import jax.numpy as jnp
from jax.experimental import pallas as pl
from jax.experimental.pallas import tpu as pltpu

G = 1000


def _head_body(g_ref, w1_ref, b1_ref, a_ref, w2_ref, b2_ref, out_ref):
    g = g_ref[...]
    h = g @ w1_ref[...] + b1_ref[...]
    h = jnp.maximum(h, 0.0) + a_ref[0, 0] * jnp.minimum(h, 0.0)
    out_ref[...] = h @ w2_ref[...] + b2_ref[...]


def _gin(xs, xd, ei, p):
    agg = jnp.zeros((xd.shape[0], xs.shape[1]), xs.dtype).at[ei[1]].add(xs[ei[0]])
    h = xd + agg
    h = h @ p["W"] + p["b"]
    h = jnp.maximum(h, 0.0) + p["a"] * jnp.minimum(h, 0.0)
    mu = h.mean(axis=0)
    var = h.var(axis=0)
    return (h - mu) / jnp.sqrt(var + 1e-5) * p["gamma"] + p["beta"]


def kernel(x_atoms, x_bonds, x_monosacchs, ei_aa, ei_ab, ei_bb, ei_bm, ei_mm,
           batch_atoms, batch_bonds, batch_monosacchs, params):
    xa = params["atom_emb"][x_atoms]
    xb = params["bond_emb"][x_bonds]
    xm = params["mono_emb"][x_monosacchs]
    for p in params["convs"]:
        na = _gin(xa, xa, ei_aa, p["aa"])
        nb = _gin(xa, xb, ei_ab, p["ab"]) + _gin(xb, xb, ei_bb, p["bb"])
        nm = _gin(xb, xm, ei_bm, p["bm"]) + _gin(xm, xm, ei_mm, p["mm"])
        xa, xb, xm = na, nb, nm
    nodes = jnp.concatenate([xa, xb, xm], axis=0)
    ids = jnp.concatenate([batch_atoms, batch_bonds, batch_monosacchs], axis=0)
    s = jax.ops.segment_sum(nodes, ids, num_segments=G)
    c = jax.ops.segment_sum(jnp.ones((nodes.shape[0],), nodes.dtype), ids, num_segments=G)
    g = s / jnp.clip(c, 1.0)[:, None]
    hd = params["head"]
    pred = pl.pallas_call(
        _head_body,
        out_shape=jax.ShapeDtypeStruct((G, 1), jnp.float32),
    )(g, hd["l1"]["W"], hd["l1"]["b"][None, :], hd["a"][None, None],
      hd["l2"]["W"], hd["l2"]["b"][None, :])
    return jnp.squeeze(pred)


# speedup vs baseline: 1.5746x; 1.5746x over previous
import functools

import jax
import jax.numpy as jnp
from jax import lax
from jax.experimental import pallas as pl
from jax.experimental.pallas import tpu as pltpu
from jax.experimental.pallas import tpu_sc as plsc

NC = 2
NS = 16
LANE = 128
NSLAB = 2
SW = 64

NA = 50000
NB = 50000
NM = 5000
PA = 51200
PB = 51200
PM = 6144
G = 1000
PG = 1024
BLK = 256

_f32 = jnp.float32
_bf16 = jnp.bfloat16


def _mesh():
    return plsc.VectorSubcoreMesh(core_axis_name="c", subcore_axis_name="s",
                                  num_cores=NC, num_subcores=NS)


def _ceil_to(x, m):
    return (x + m - 1) // m * m


GRP = 8


def _zch(pd):
    rpt = pd // NS
    return next(d for d in range(80, 0, -8) if rpt % d == 0)


@functools.cache
def _agg_kernel(epad, ps_src, pd):
    nbt = epad // LANE // NS
    ngrp = nbt // GRP
    rpt = pd // NS
    zch = _zch(pd)
    assert nbt % GRP == 0

    def body(src_off, dst2d, zrows, xflat, out, src_g, dst_v, rows, zbuf, acc,
             sem_a, sem_b, sem_sa, sem_sb):
        c = lax.axis_index("c")
        t = lax.axis_index("s")
        row0 = t * rpt
        pltpu.sync_copy(zrows, zbuf)
        pltpu.sync_copy(dst2d.at[t], dst_v)
        slab = c
        for z in range(rpt // zch):
            pltpu.sync_copy(zbuf, acc.at[pl.ds(row0 + z * zch, zch)])
        plsc.subcore_barrier()

        def src_cp(g, b, sem):
            return pltpu.make_async_copy(
                src_off.at[slab].at[t].at[pl.ds(g * GRP, GRP)],
                src_g.at[b], sem)

        def inner8(g, b):
            sg = src_g.at[b]

            def gcp(j, r, sem):
                return pltpu.make_async_copy(xflat.at[sg.at[j]],
                                             rows.at[r], sem)
            gcp(0, 0, sem_a).start()
            for j in range(GRP):
                r, sem = j % 2, (sem_a, sem_b)[j % 2]
                if j + 1 < GRP:
                    gcp(j + 1, (j + 1) % 2,
                        (sem_a, sem_b)[(j + 1) % 2]).start()
                gcp(j, r, sem).wait()
                pltpu.sync_copy(rows.at[r],
                                acc.at[dst_v.at[g * GRP + j]], add=True)

        src_cp(0, 0, sem_sa).start()

        def gstep(g2, _):
            g0 = g2 * 2

            @pl.when(g0 + 1 < ngrp)
            def _():
                src_cp(g0 + 1, 1, sem_sb).start()
            src_cp(g0, 0, sem_sa).wait()
            inner8(g0, 0)

            @pl.when(g0 + 2 < ngrp)
            def _():
                src_cp(g0 + 2, 0, sem_sa).start()

            @pl.when(g0 + 1 < ngrp)
            def _():
                src_cp(g0 + 1, 1, sem_sb).wait()
                inner8(g0 + 1, 1)
            return 0

        lax.fori_loop(0, (ngrp + 1) // 2, gstep, 0)
        plsc.subcore_barrier()
        for z in range(rpt // zch):
            sl = pl.ds(row0 + z * zch, zch)
            pltpu.sync_copy(acc.at[sl], out.at[slab].at[sl])

    return pl.kernel(
        body,
        out_type=jax.ShapeDtypeStruct((NSLAB, pd, SW), _bf16),
        mesh=_mesh(),
        compiler_params=pltpu.CompilerParams(use_tc_tiling_on_sc=False),
        scratch_types=[
            pltpu.VMEM((2, GRP, LANE), jnp.int32),
            pltpu.VMEM((nbt, LANE), jnp.int32),
            pltpu.VMEM((2, LANE, SW), _bf16),
            pltpu.VMEM((zch, SW), _bf16),
            pltpu.VMEM_SHARED((pd, SW), _bf16),
            pltpu.SemaphoreType.DMA,
            pltpu.SemaphoreType.DMA,
            pltpu.SemaphoreType.DMA,
            pltpu.SemaphoreType.DMA,
        ],
    )


@functools.cache
def _embed_kernel(ppad, tabp):
    n = ppad // BLK

    def body(ids_ref, tab_ref, out_ref):
        ids = ids_ref[0]
        ioti = lax.broadcasted_iota(jnp.int32, (tabp, BLK), 0)
        oh = (ioti == ids).astype(_f32)
        x = lax.dot_general(oh, tab_ref[...], (((0,), (0,)), ((), ())),
                            preferred_element_type=_f32)
        xb = x.astype(_bf16)
        for s in range(NSLAB):
            out_ref[s] = xb[:, SW * s:SW * (s + 1)]

    return pl.pallas_call(
        body,
        grid=(n,),
        in_specs=[
            pl.BlockSpec((1, 1, BLK), lambda i: (i, 0, 0)),
            pl.BlockSpec((tabp, 128), lambda i: (0, 0)),
        ],
        out_specs=pl.BlockSpec((NSLAB, BLK, SW), lambda i: (0, i, 0)),
        out_shape=jax.ShapeDtypeStruct((NSLAB, ppad, SW), _bf16),
    )


@functools.cache
def _mm_kernel(pd, nd):
    n = pd // BLK

    def body(xd_ref, agg_ref, w_ref, b_ref, a_ref, h_ref, st_ref, acc_ref):
        i = pl.program_id(0)
        xs = xd_ref[...].astype(_f32) + agg_ref[...].astype(_f32)
        h = jnp.zeros((BLK, 128), _f32)
        for s in range(NSLAB):
            h = h + jnp.dot(xs[s], w_ref[SW * s:SW * (s + 1), :],
                            preferred_element_type=_f32)
        h = h + b_ref[...]
        h = jnp.maximum(h, 0.0) + a_ref[...] * jnp.minimum(h, 0.0)
        h_ref[...] = h
        rows = i * BLK + lax.broadcasted_iota(jnp.int32, (BLK, 128), 0)
        hm = jnp.where(rows < nd, h, 0.0)

        @pl.when(i == 0)
        def _():
            acc_ref[...] = jnp.zeros((8, 128), _f32)
        acc_ref[0:1, :] += jnp.sum(hm, axis=0, keepdims=True)
        acc_ref[1:2, :] += jnp.sum(hm * hm, axis=0, keepdims=True)

        @pl.when(i == n - 1)
        def _():
            st_ref[...] = acc_ref[...]

    return pl.pallas_call(
        body,
        grid=(n,),
        in_specs=[
            pl.BlockSpec((NSLAB, BLK, SW), lambda i: (0, i, 0)),
            pl.BlockSpec((NSLAB, BLK, SW), lambda i: (0, i, 0)),
            pl.BlockSpec((128, 128), lambda i: (0, 0)),
            pl.BlockSpec((1, 128), lambda i: (0, 0)),
            pl.BlockSpec((1, 1), lambda i: (0, 0)),
        ],
        out_specs=[
            pl.BlockSpec((BLK, 128), lambda i: (i, 0)),
            pl.BlockSpec((8, 128), lambda i: (0, 0)),
        ],
        out_shape=[
            jax.ShapeDtypeStruct((pd, 128), _f32),
            jax.ShapeDtypeStruct((8, 128), _f32),
        ],
        scratch_shapes=[pltpu.VMEM((8, 128), _f32)],
    )


@functools.cache
def _combine_kernel(pd, nd, two, with_pool):
    n = pd // BLK
    inv_n = 1.0 / nd

    def norm(h_ref, st_ref, g_ref, bt_ref):
        mu = st_ref[0:1, :] * inv_n
        var = st_ref[1:2, :] * inv_n - mu * mu
        s = g_ref[...] * lax.rsqrt(var + 1e-5)
        t = bt_ref[...] - mu * s
        return h_ref[...] * s + t

    def body(*refs):
        refs = list(refs)
        scr = []
        if with_pool:
            pacc_ref, cacc_ref = refs[-2:]
            scr = refs[-2:]
            refs = refs[:-2]
        if with_pool:
            ids_ref = refs.pop(0)
        if two:
            h1, st1, g1, bt1, h2, st2, g2, bt2 = refs[:8]
            out_refs = refs[8:]
            y = norm(h1, st1, g1, bt1) + norm(h2, st2, g2, bt2)
        else:
            h1, st1, g1, bt1 = refs[:4]
            out_refs = refs[4:]
            y = norm(h1, st1, g1, bt1)
        out_ref = out_refs[0]
        yb = y.astype(_bf16)
        for s in range(NSLAB):
            out_ref[s] = yb[:, SW * s:SW * (s + 1)]
        if with_pool:
            i = pl.program_id(0)
            pool_ref, cnt_ref = out_refs[1], out_refs[2]
            ids = ids_ref[0]
            iot = lax.broadcasted_iota(jnp.int32, (PG, BLK), 0)
            oh = (iot == ids).astype(_f32)

            @pl.when(i == 0)
            def _():
                pacc_ref[...] = jnp.zeros((PG, 128), _f32)
                cacc_ref[...] = jnp.zeros((PG, 8), _f32)
            pacc_ref[...] += lax.dot_general(
                oh, y, (((1,), (0,)), ((), ())), preferred_element_type=_f32)
            cacc_ref[...] += lax.dot_general(
                oh, jnp.ones((BLK, 8), _f32), (((1,), (0,)), ((), ())),
                preferred_element_type=_f32)

            @pl.when(i == n - 1)
            def _():
                pool_ref[...] = pacc_ref[...]
                cnt_ref[...] = cacc_ref[...]

    hspec = pl.BlockSpec((BLK, 128), lambda i: (i, 0))
    sspec = pl.BlockSpec((8, 128), lambda i: (0, 0))
    pspec = pl.BlockSpec((1, 128), lambda i: (0, 0))
    ins = [hspec, sspec, pspec, pspec]
    if two:
        ins = ins + ins
    outs = [pl.BlockSpec((NSLAB, BLK, SW), lambda i: (0, i, 0))]
    oshapes = [jax.ShapeDtypeStruct((NSLAB, pd, SW), _bf16)]
    scratch = []
    if with_pool:
        ins = [pl.BlockSpec((1, 1, BLK), lambda i: (i, 0, 0))] + ins
        outs += [pl.BlockSpec((PG, 128), lambda i: (0, 0)),
                 pl.BlockSpec((PG, 8), lambda i: (0, 0))]
        oshapes += [jax.ShapeDtypeStruct((PG, 128), _f32),
                    jax.ShapeDtypeStruct((PG, 8), _f32)]
        scratch = [pltpu.VMEM((PG, 128), _f32), pltpu.VMEM((PG, 8), _f32)]
    return pl.pallas_call(
        body,
        grid=(n,),
        in_specs=ins,
        out_specs=outs,
        out_shape=oshapes,
        scratch_shapes=scratch,
    )


def _head_kernel():
    def body(pa_ref, ca_ref, pb_ref, cb_ref, pm_ref, cm_ref,
             w1_ref, b1_ref, a_ref, w2_ref, b2_ref, out_ref):
        pool = pa_ref[...] + pb_ref[...] + pm_ref[...]
        cnt = ca_ref[:, 0:1] + cb_ref[:, 0:1] + cm_ref[:, 0:1]
        g = pool / jnp.maximum(cnt, 1.0)
        h = jnp.dot(g, w1_ref[...], preferred_element_type=_f32)
        h = h + b1_ref[...]
        h = jnp.maximum(h, 0.0) + a_ref[...] * jnp.minimum(h, 0.0)
        out_ref[...] = jnp.dot(h, w2_ref[...],
                               preferred_element_type=_f32) + b2_ref[...]

    return pl.pallas_call(
        body,
        out_shape=jax.ShapeDtypeStruct((PG, 8), _f32),
    )


def _pad_nodes(idx, ppad, fill):
    return jnp.concatenate(
        [idx.astype(jnp.int32),
         jnp.full((ppad - idx.shape[0],), fill, jnp.int32)])


def _off_stack(idx, step):
    return jnp.stack([idx + s * step for s in range(NSLAB)]).reshape(
        NSLAB, NS, -1, LANE)


def _prep_edges(ei, n_src, p_src, n_dst):
    e = ei.shape[1]
    epad = _ceil_to(e, NS * LANE * GRP)
    src = jnp.concatenate([ei[0].astype(jnp.int32),
                           jnp.zeros((epad - e,), jnp.int32)])
    dst = jnp.concatenate([ei[1].astype(jnp.int32),
                           jnp.full((epad - e,), n_dst, jnp.int32)])
    return _off_stack(src, p_src), dst.reshape(NS, -1, LANE), epad


def kernel(x_atoms, x_bonds, x_monosacchs, ei_aa, ei_ab, ei_bb, ei_bm, ei_mm,
           batch_atoms, batch_bonds, batch_monosacchs, params):
    z_big = jnp.zeros((_zch(PA), SW), _bf16)
    z_sml = jnp.zeros((_zch(PM), SW), _bf16)

    def embed(idx, ppad, emb, tabp):
        tab = jnp.pad(emb, ((0, tabp - emb.shape[0]), (0, 0)))
        ids = _pad_nodes(idx, ppad, 0).reshape(ppad // BLK, 1, BLK)
        return _embed_kernel(ppad, tabp)(ids, tab)

    xa = embed(x_atoms, PA, params["atom_emb"], 128)
    xb = embed(x_bonds, PB, params["bond_emb"], 128)
    xm = embed(x_monosacchs, PM, params["mono_emb"], 256)

    so_aa, d2_aa, ep_aa = _prep_edges(ei_aa, NA, PA, NA)
    so_ab, d2_ab, ep_ab = _prep_edges(ei_ab, NA, PA, NB)
    so_bb, d2_bb, ep_bb = _prep_edges(ei_bb, NB, PB, NB)
    so_bm, d2_bm, ep_bm = _prep_edges(ei_bm, NB, PB, NM)
    so_mm, d2_mm, ep_mm = _prep_edges(ei_mm, NM, PM, NM)

    ids_a = _pad_nodes(batch_atoms, PA, G).reshape(PA // BLK, 1, BLK)
    ids_b = _pad_nodes(batch_bonds, PB, G).reshape(PB // BLK, 1, BLK)
    ids_m = _pad_nodes(batch_monosacchs, PM, G).reshape(PM // BLK, 1, BLK)

    pools = []
    for li, p in enumerate(params["convs"]):
        last = li == len(params["convs"]) - 1
        xaf = xa.reshape(NSLAB * PA, SW)
        xbf = xb.reshape(NSLAB * PB, SW)
        xmf = xm.reshape(NSLAB * PM, SW)
        agg_aa = _agg_kernel(ep_aa, PA, PA)(so_aa, d2_aa, z_big, xaf)
        agg_ab = _agg_kernel(ep_ab, PA, PB)(so_ab, d2_ab, z_big, xaf)
        agg_bb = _agg_kernel(ep_bb, PB, PB)(so_bb, d2_bb, z_big, xbf)
        agg_bm = _agg_kernel(ep_bm, PB, PM)(so_bm, d2_bm, z_sml, xbf)
        agg_mm = _agg_kernel(ep_mm, PM, PM)(so_mm, d2_mm, z_sml, xmf)

        def mm(pd, nd, xd, agg, pr):
            return _mm_kernel(pd, nd)(xd, agg, pr["W"], pr["b"][None, :],
                                      jnp.reshape(pr["a"], (1, 1)))

        h_aa, st_aa = mm(PA, NA, xa, agg_aa, p["aa"])
        h_ab, st_ab = mm(PB, NB, xb, agg_ab, p["ab"])
        h_bb, st_bb = mm(PB, NB, xb, agg_bb, p["bb"])
        h_bm, st_bm = mm(PM, NM, xm, agg_bm, p["bm"])
        h_mm, st_mm = mm(PM, NM, xm, agg_mm, p["mm"])

        def nparams(pr):
            return (pr["gamma"][None, :], pr["beta"][None, :])

        if last:
            xa, pl_a, cn_a = _combine_kernel(PA, NA, False, True)(
                ids_a, h_aa, st_aa, *nparams(p["aa"]))
            xb, pl_b, cn_b = _combine_kernel(PB, NB, True, True)(
                ids_b, h_ab, st_ab, *nparams(p["ab"]),
                h_bb, st_bb, *nparams(p["bb"]))
            xm, pl_m, cn_m = _combine_kernel(PM, NM, True, True)(
                ids_m, h_bm, st_bm, *nparams(p["bm"]),
                h_mm, st_mm, *nparams(p["mm"]))
            pools = [pl_a, cn_a, pl_b, cn_b, pl_m, cn_m]
        else:
            (xa,) = _combine_kernel(PA, NA, False, False)(
                h_aa, st_aa, *nparams(p["aa"]))
            (xb,) = _combine_kernel(PB, NB, True, False)(
                h_ab, st_ab, *nparams(p["ab"]), h_bb, st_bb, *nparams(p["bb"]))
            (xm,) = _combine_kernel(PM, NM, True, False)(
                h_bm, st_bm, *nparams(p["bm"]), h_mm, st_mm, *nparams(p["mm"]))

    hd = params["head"]
    w2 = jnp.pad(hd["l2"]["W"], ((0, 0), (0, 7)))
    b2 = jnp.pad(hd["l2"]["b"], (0, 7))[None, :]
    out = _head_kernel()(*pools, hd["l1"]["W"], hd["l1"]["b"][None, :],
                         jnp.reshape(hd["a"], (1, 1)), w2, b2)
    return out[:G, 0]
